# baseline (speedup 1.0000x reference)
import functools

import jax
import jax.numpy as jnp
from jax import lax
from jax.experimental import pallas as pl
from jax.experimental.pallas import tpu as pltpu

N_BINS = 15
CHUNK_ROWS = 512
NBUF = 8


def _process(x16, labels, cnt_ref, sconf_ref, sacc_ref):
    x = x16.astype(jnp.float32)
    m = jnp.max(x, axis=1, keepdims=True)
    s = jnp.sum(jnp.exp(x - m), axis=1)
    conf = 1.0 / s
    col = lax.broadcasted_iota(jnp.int32, x.shape, 1)
    hit = (x == m) & (col == labels[:, None])
    acc = jnp.max(hit.astype(jnp.float32), axis=1)

    k = lax.broadcasted_iota(jnp.int32, (1, N_BINS), 1).astype(jnp.float32)
    lo = k / N_BINS
    hi = (k + 1.0) / N_BINS
    c2 = conf[:, None]
    mask = ((c2 > lo) & (c2 <= hi)).astype(jnp.float32)
    cnt_ref[...] += jnp.sum(mask, axis=0)
    sconf_ref[...] += jnp.sum(c2 * mask, axis=0)
    sacc_ref[...] += jnp.sum(acc[:, None] * mask, axis=0)


def _ece_body(logits_hbm, labels_ref, ece_ref, bufs, sems,
              cnt_ref, sconf_ref, sacc_ref, *, n_total, n_macro):
    j = pl.program_id(0)

    def start(b, chunk):
        pltpu.make_async_copy(
            logits_hbm.at[pl.ds(chunk * CHUNK_ROWS, CHUNK_ROWS), :],
            bufs.at[b], sems.at[b]).start()

    @pl.when(j == 0)
    def _prologue():
        cnt_ref[...] = jnp.zeros_like(cnt_ref)
        sconf_ref[...] = jnp.zeros_like(sconf_ref)
        sacc_ref[...] = jnp.zeros_like(sacc_ref)
        for b in range(NBUF):
            start(b, b)

    for b in range(NBUF):
        pltpu.make_async_copy(
            logits_hbm.at[pl.ds(0, CHUNK_ROWS), :], bufs.at[b], sems.at[b]
        ).wait()
        _process(bufs[b], labels_ref[0, b], cnt_ref, sconf_ref, sacc_ref)

        @pl.when(j < n_macro - 1)
        def _refill():
            start(b, (j + 1) * NBUF + b)

    @pl.when(j == n_macro - 1)
    def _finish():
        cnt = cnt_ref[...]
        safe = jnp.maximum(cnt, 1.0)
        gap = jnp.abs(sconf_ref[...] / safe - sacc_ref[...] / safe) * (cnt / n_total)
        gap = jnp.where(cnt > 0, gap, 0.0)
        ece_ref[...] = jnp.sum(gap, keepdims=True)


def kernel(logits, labels):
    n, c = logits.shape
    labels = labels.astype(jnp.int32)
    n_macro = n // (CHUNK_ROWS * NBUF)
    labels3 = labels.reshape(n_macro, NBUF, CHUNK_ROWS)
    return pl.pallas_call(
        functools.partial(_ece_body, n_total=float(n), n_macro=n_macro),
        grid=(n_macro,),
        in_specs=[
            pl.BlockSpec(memory_space=pltpu.MemorySpace.HBM),
            pl.BlockSpec((1, NBUF, CHUNK_ROWS), lambda j: (j, 0, 0)),
        ],
        out_specs=pl.BlockSpec((1,), lambda j: (0,)),
        out_shape=jax.ShapeDtypeStruct((1,), jnp.float32),
        scratch_shapes=[
            pltpu.VMEM((NBUF, CHUNK_ROWS, c), jnp.bfloat16),
            pltpu.SemaphoreType.DMA((NBUF,)),
            pltpu.VMEM((N_BINS,), jnp.float32),
            pltpu.VMEM((N_BINS,), jnp.float32),
            pltpu.VMEM((N_BINS,), jnp.float32),
        ],
    )(logits.astype(jnp.bfloat16), labels3)


# speedup vs baseline: 1.0448x; 1.0448x over previous
import functools

import jax
import jax.numpy as jnp
from jax import lax
from jax.experimental import pallas as pl
from jax.experimental.pallas import tpu as pltpu
from jax.experimental.pallas import tpu_sc as plsc

N_BINS = 15
BLOCK_ROWS = 4096
N_WORKERS = 32
LANES = 16


def _scan_body(logits_ref, labels_ref, conf_ref, acc_ref):
    x = logits_ref[...]
    m = jnp.max(x, axis=1, keepdims=True)
    s = jnp.sum(jnp.exp(x - m), axis=1)
    conf_ref[...] = 1.0 / s
    col = lax.broadcasted_iota(jnp.int32, x.shape, 1)
    hit = (x == m) & (col == labels_ref[...][:, None])
    acc_ref[...] = jnp.max(hit.astype(jnp.float32), axis=1)


def _tc_scan(logits, labels):
    n, c = logits.shape
    return pl.pallas_call(
        _scan_body,
        grid=(n // BLOCK_ROWS,),
        in_specs=[
            pl.BlockSpec((BLOCK_ROWS, c), lambda i: (i, 0)),
            pl.BlockSpec((BLOCK_ROWS,), lambda i: (i,)),
        ],
        out_specs=[
            pl.BlockSpec((BLOCK_ROWS,), lambda i: (i,)),
            pl.BlockSpec((BLOCK_ROWS,), lambda i: (i,)),
        ],
        out_shape=[
            jax.ShapeDtypeStruct((n,), jnp.float32),
            jax.ShapeDtypeStruct((n,), jnp.float32),
        ],
    )(logits, labels)


def _sc_bin(conf, acc):
    n = conf.shape[0]
    chunk = n // N_WORKERS
    mesh = plsc.VectorSubcoreMesh(core_axis_name="c", subcore_axis_name="s")

    @functools.partial(
        pl.kernel,
        mesh=mesh,
        out_type=jax.ShapeDtypeStruct((N_WORKERS, 3 * N_BINS, LANES),
                                      jnp.float32),
        scratch_types=[
            pltpu.VMEM((chunk,), jnp.float32),
            pltpu.VMEM((chunk,), jnp.float32),
            pltpu.VMEM((3 * N_BINS, LANES), jnp.float32),
        ],
    )
    def sc_kernel(conf_hbm, acc_hbm, out_hbm, confbuf, accbuf, part):
        w = lax.axis_index("s") * 2 + lax.axis_index("c")
        pltpu.sync_copy(conf_hbm.at[pl.ds(w * chunk, chunk)], confbuf)
        pltpu.sync_copy(acc_hbm.at[pl.ds(w * chunk, chunk)], accbuf)
        zero = jnp.zeros((LANES,), jnp.float32)
        for r in range(3 * N_BINS):
            part[r] = zero

        def step(i, carry):
            cv = confbuf[pl.ds(i * LANES, LANES)]
            av = accbuf[pl.ds(i * LANES, LANES)]
            for b in range(N_BINS):
                in_bin = (cv > jnp.float32(b / N_BINS)) & (
                    cv <= jnp.float32((b + 1) / N_BINS))
                mf = jnp.where(in_bin, 1.0, 0.0)
                plsc.addupdate(part.at[b], mf)
                plsc.addupdate(part.at[N_BINS + b], cv * mf)
                plsc.addupdate(part.at[2 * N_BINS + b], av * mf)
            return carry

        lax.fori_loop(0, chunk // LANES, step, 0)
        pltpu.sync_copy(part, out_hbm.at[w])

    return sc_kernel(conf, acc)


def _combine_body(part_ref, ece_ref, *, n_total):
    x = part_ref[...]
    lane = jnp.sum(x, axis=2)
    tot = jnp.sum(lane, axis=0)
    cnt = tot[0:N_BINS]
    sconf = tot[N_BINS:2 * N_BINS]
    sacc = tot[2 * N_BINS:3 * N_BINS]
    safe = jnp.maximum(cnt, 1.0)
    gap = jnp.abs(sconf / safe - sacc / safe) * (cnt / n_total)
    gap = jnp.where(cnt > 0, gap, 0.0)
    ece_ref[...] = jnp.sum(gap, keepdims=True)


def _tc_combine(partials, n_total):
    return pl.pallas_call(
        functools.partial(_combine_body, n_total=n_total),
        out_shape=jax.ShapeDtypeStruct((1,), jnp.float32),
    )(partials)


def kernel(logits, labels):
    n, _ = logits.shape
    conf, acc = _tc_scan(logits, labels.astype(jnp.int32))
    partials = _sc_bin(conf, acc)
    return _tc_combine(partials, float(n))


# speedup vs baseline: 1.0481x; 1.0031x over previous
import functools

import jax
import jax.numpy as jnp
from jax import lax
from jax.experimental import pallas as pl
from jax.experimental.pallas import tpu as pltpu
from jax.experimental.pallas import tpu_sc as plsc

N_BINS = 15
BLOCK_ROWS = 2048
N_WORKERS = 32
LANES = 16


def _scan_body(logits_ref, labels_ref, conf_ref, acc_ref):
    x = logits_ref[...]
    m = jnp.max(x, axis=1, keepdims=True)
    s = jnp.sum(jnp.exp(x - m), axis=1)
    conf_ref[...] = 1.0 / s
    col = lax.broadcasted_iota(jnp.int32, x.shape, 1)
    hit = (x == m) & (col == labels_ref[...][:, None])
    acc_ref[...] = jnp.max(hit.astype(jnp.float32), axis=1)


def _tc_scan(logits, labels):
    n, c = logits.shape
    return pl.pallas_call(
        _scan_body,
        grid=(n // BLOCK_ROWS,),
        in_specs=[
            pl.BlockSpec((BLOCK_ROWS, c), lambda i: (i, 0)),
            pl.BlockSpec((BLOCK_ROWS,), lambda i: (i,)),
        ],
        out_specs=[
            pl.BlockSpec((BLOCK_ROWS,), lambda i: (i,)),
            pl.BlockSpec((BLOCK_ROWS,), lambda i: (i,)),
        ],
        out_shape=[
            jax.ShapeDtypeStruct((n,), jnp.float32),
            jax.ShapeDtypeStruct((n,), jnp.float32),
        ],
    )(logits, labels)


def _sc_bin(conf, acc):
    n = conf.shape[0]
    chunk = n // N_WORKERS
    mesh = plsc.VectorSubcoreMesh(core_axis_name="c", subcore_axis_name="s")

    @functools.partial(
        pl.kernel,
        mesh=mesh,
        out_type=jax.ShapeDtypeStruct((N_WORKERS, 3 * N_BINS, LANES),
                                      jnp.float32),
        scratch_types=[
            pltpu.VMEM((chunk,), jnp.float32),
            pltpu.VMEM((chunk,), jnp.float32),
            pltpu.VMEM((3 * N_BINS, LANES), jnp.float32),
        ],
    )
    def sc_kernel(conf_hbm, acc_hbm, out_hbm, confbuf, accbuf, part):
        w = lax.axis_index("s") * 2 + lax.axis_index("c")
        pltpu.sync_copy(conf_hbm.at[pl.ds(w * chunk, chunk)], confbuf)
        pltpu.sync_copy(acc_hbm.at[pl.ds(w * chunk, chunk)], accbuf)
        zero = jnp.zeros((LANES,), jnp.float32)
        for r in range(3 * N_BINS):
            part[r] = zero

        def step(i, carry):
            cv = confbuf[pl.ds(i * LANES, LANES)]
            av = accbuf[pl.ds(i * LANES, LANES)]
            for b in range(N_BINS):
                in_bin = (cv > jnp.float32(b / N_BINS)) & (
                    cv <= jnp.float32((b + 1) / N_BINS))
                mf = jnp.where(in_bin, 1.0, 0.0)
                plsc.addupdate(part.at[b], mf)
                plsc.addupdate(part.at[N_BINS + b], cv * mf)
                plsc.addupdate(part.at[2 * N_BINS + b], av * mf)
            return carry

        lax.fori_loop(0, chunk // LANES, step, 0)
        pltpu.sync_copy(part, out_hbm.at[w])

    return sc_kernel(conf, acc)


def _combine_body(part_ref, ece_ref, *, n_total):
    x = part_ref[...]
    lane = jnp.sum(x, axis=2)
    tot = jnp.sum(lane, axis=0)
    cnt = tot[0:N_BINS]
    sconf = tot[N_BINS:2 * N_BINS]
    sacc = tot[2 * N_BINS:3 * N_BINS]
    safe = jnp.maximum(cnt, 1.0)
    gap = jnp.abs(sconf / safe - sacc / safe) * (cnt / n_total)
    gap = jnp.where(cnt > 0, gap, 0.0)
    ece_ref[...] = jnp.sum(gap, keepdims=True)


def _tc_combine(partials, n_total):
    return pl.pallas_call(
        functools.partial(_combine_body, n_total=n_total),
        out_shape=jax.ShapeDtypeStruct((1,), jnp.float32),
    )(partials)


def kernel(logits, labels):
    n, _ = logits.shape
    conf, acc = _tc_scan(logits, labels.astype(jnp.int32))
    partials = _sc_bin(conf, acc)
    return _tc_combine(partials, float(n))
